# baseline (speedup 1.0000x reference)
import functools

import jax
import jax.numpy as jnp
from jax import lax
from jax.experimental import pallas as pl
from jax.experimental.pallas import tpu as pltpu
from jax.experimental.pallas import tpu_sc as plsc


@functools.lru_cache(maxsize=None)
def _make_gather(V, D, B):
    info = plsc.get_sparse_core_info()
    NC, NS, L = info.num_cores, info.num_subcores, info.num_lanes
    NW = NC * NS
    assert B % (8 * NW) == 0 and D == 64 and V % 2 == 0
    b_per_w = B // NW
    n_groups = b_per_w // L
    mesh = plsc.VectorSubcoreMesh(core_axis_name="c", subcore_axis_name="s")

    @functools.partial(
        pl.kernel,
        mesh=mesh,
        compiler_params=pltpu.CompilerParams(needs_layout_passes=False),
        out_type=jax.ShapeDtypeStruct((B, 2 * D), jnp.float32),
        scratch_types=[
            pltpu.VMEM((b_per_w,), jnp.int32),
            pltpu.VMEM((b_per_w,), jnp.int32),
            pltpu.VMEM((b_per_w, 2 * D), jnp.float32),
            pltpu.SemaphoreType.DMA,
        ],
    )
    def k(t2_hbm, idx_hbm, out_hbm, lab_v, pidx_v, pair_v, sem):
        wid = lax.axis_index("s") * NC + lax.axis_index("c")
        base = wid * b_per_w
        pltpu.sync_copy(idx_hbm.at[pl.ds(base, b_per_w)], lab_v)
        for g in range(n_groups):
            lab = lab_v[pl.ds(g * L, L)]
            pidx_v[pl.ds(g * L, L)] = lab >> 1
        pltpu.async_copy(t2_hbm.at[pidx_v], pair_v, sem).wait()

        def select(g, carry):
            rows = lax.iota(jnp.int32, L) + g * L
            half = (lab_v[pl.ds(g * L, L)] & 1) << 6
            for d in range(D):
                vals = plsc.load_gather(pair_v, [rows, half + d])
                plsc.store_scatter(pair_v, [rows, jnp.full((L,), d, jnp.int32)], vals)
            return carry

        lax.fori_loop(0, n_groups, select, 0)
        pltpu.sync_copy(pair_v, out_hbm.at[pl.ds(base, b_per_w)])

    return k


def kernel(labels, table):
    B, = labels.shape
    V, D = table.shape
    t2 = jnp.reshape(table, (V // 2, 2 * D))
    out128 = _make_gather(V, D, B)(t2, labels.astype(jnp.int32))
    return out128[:, :D]


# speedup vs baseline: 1.1723x; 1.1723x over previous
import functools

import jax
import jax.numpy as jnp
from jax import lax
from jax.experimental import pallas as pl
from jax.experimental.pallas import tpu as pltpu
from jax.experimental.pallas import tpu_sc as plsc


@functools.lru_cache(maxsize=None)
def _make_gather(V, D, B):
    info = plsc.get_sparse_core_info()
    NC, NS = info.num_cores, info.num_subcores
    NW = NC * NS
    assert B % (8 * NW) == 0
    b_per_w = B // NW
    mesh = plsc.VectorSubcoreMesh(core_axis_name="c", subcore_axis_name="s")

    @functools.partial(
        pl.kernel,
        mesh=mesh,
        out_type=jax.ShapeDtypeStruct((B, 2 * D), jnp.float32),
        scratch_types=[
            pltpu.VMEM((b_per_w,), jnp.int32),
            pltpu.VMEM((b_per_w, 2 * D), jnp.float32),
            pltpu.SemaphoreType.DMA,
        ],
    )
    def k(tbl_hbm, idx_hbm, out_hbm, lab_v, rows_v, sem):
        wid = lax.axis_index("s") * NC + lax.axis_index("c")
        base = wid * b_per_w
        pltpu.sync_copy(idx_hbm.at[pl.ds(base, b_per_w)], lab_v)
        pltpu.async_copy(tbl_hbm.at[lab_v], rows_v, sem).wait()
        pltpu.sync_copy(rows_v, out_hbm.at[pl.ds(base, b_per_w)])

    return k


def kernel(labels, table):
    B, = labels.shape
    V, D = table.shape
    tp = jnp.pad(table, ((0, 0), (0, D)))
    out128 = _make_gather(V, D, B)(tp, labels.astype(jnp.int32))
    return out128[:, :D]
